# baseline (speedup 1.0000x reference)
import functools

import jax
import jax.numpy as jnp
from jax import lax
from jax.experimental import pallas as pl
from jax.experimental.pallas import tpu as pltpu
from jax.experimental.pallas import tpu_sc as plsc

VOCAB = 1000
T_DIM = 200
B_DIM = 4096
NTOK = T_DIM * B_DIM

NC, NS, L = 2, 16, 16
NW = NC * NS

CS = 32
NSUB = 4
CW = CS * NSUB
NSLAB = 8
NTG = NW // NSLAB
TPG = T_DIM // NTG
BBLK = 512
NBBLK = B_DIM // BBLK
NB16 = BBLK // L

LB = B_DIM // NW
NG = 13
NGA = 8


def _lse_body(t_ref, o_ref):
    x = t_ref[...]
    m = jnp.max(x, axis=1, keepdims=True)
    o_ref[...] = m + jnp.log(jnp.sum(jnp.exp(x - m), axis=1, keepdims=True))


_MESH = plsc.VectorSubcoreMesh(
    core_axis_name="c", subcore_axis_name="s", num_cores=NC, num_subcores=NS
)


@functools.partial(
    pl.kernel,
    out_type=[
        jax.ShapeDtypeStruct((T_DIM * VOCAB, B_DIM), jnp.float32),
        jax.ShapeDtypeStruct((NW, L), jnp.float32),
    ],
    mesh=_MESH,
    compiler_params=pltpu.CompilerParams(
        needs_layout_passes=False, use_tc_tiling_on_sc=False
    ),
    scratch_types=[
        pltpu.VMEM((VOCAB, CS), jnp.float32),
        pltpu.VMEM((CS, BBLK), jnp.float32),
        pltpu.VMEM((CS, BBLK), jnp.float32),
        pltpu.VMEM((B_DIM,), jnp.int32),
        pltpu.VMEM((B_DIM,), jnp.int32),
        pltpu.VMEM((VOCAB,), jnp.float32),
        pltpu.VMEM((T_DIM, LB), jnp.int32),
        pltpu.VMEM((NG * L,), jnp.int32),
        pltpu.VMEM((NG * L,), jnp.int32),
        pltpu.VMEM((NGA * L,), jnp.int32),
        pltpu.VMEM((NGA * L,), jnp.int32),
        pltpu.VMEM(((NG - NGA) * L,), jnp.int32),
        pltpu.VMEM(((NG - NGA) * L,), jnp.int32),
        pltpu.VMEM((NGA * L,), jnp.float32),
        pltpu.VMEM((NGA * L,), jnp.float32),
        pltpu.VMEM(((NG - NGA) * L,), jnp.float32),
        pltpu.VMEM(((NG - NGA) * L,), jnp.float32),
        pltpu.VMEM((L,), jnp.float32),
        pltpu.SemaphoreType.DMA,
        pltpu.SemaphoreType.DMA,
        pltpu.SemaphoreType.DMA,
        pltpu.SemaphoreType.DMA,
        pltpu.SemaphoreType.DMA,
        pltpu.SemaphoreType.DMA,
        pltpu.SemaphoreType.DMA,
        pltpu.SemaphoreType.DMA,
    ],
)
def _sc_gather_loss(idx_hbm, tgtf_hbm, table_hbm, tablef_hbm, lse_hbm,
                    out_hbm, part_hbm,
                    slab_v, ov0, ov1, ir0, ir1, lse_v, idxs_v,
                    tg0, tg1, pa0, pa1, pb0, pb1, va0, va1, vb0, vb1, acc_v,
                    isem0, isem1, osem0, osem1, tsem0, tsem1, psem0, psem1):
    wid = lax.axis_index("s") * NC + lax.axis_index("c")
    c0b = (wid % NSLAB) * CW
    t0 = (wid // NSLAB) * TPG

    irs, ovs = (ir0, ir1), (ov0, ov1)
    isems, osems = (isem0, isem1), (osem0, osem1)
    tgs, tsems = (tg0, tg1), (tsem0, tsem1)
    pas, pbs = (pa0, pa1), (pb0, pb1)
    vas, vbs = (va0, va1), (vb0, vb1)
    psems = (psem0, psem1)
    iota16 = lax.iota(jnp.int32, L)

    pltpu.sync_copy(lse_hbm, lse_v)

    def fire_idx(slot, t):
        pltpu.async_copy(idx_hbm.at[t], irs[slot], isems[slot])

    def wait_idx(slot):
        pltpu.make_async_copy(idx_hbm.at[0], irs[slot], isems[slot]).wait()

    def fire_out(slot, r0, b0):
        pltpu.async_copy(
            ovs[slot], out_hbm.at[pl.ds(r0, CS), pl.ds(b0, BBLK)], osems[slot]
        )

    def wait_out(slot):
        pltpu.make_async_copy(
            ovs[slot], out_hbm.at[pl.ds(0, CS), pl.ds(0, BBLK)], osems[slot]
        ).wait()

    def process_t(ts, t, c0, skip_first):
        r0 = t * VOCAB + c0

        def bpair(k, carry):
            for p in (0, 1):
                bblk = 2 * k + p
                if ts == 0:
                    @pl.when(
                        jnp.logical_not(jnp.logical_and(skip_first, k == 0))
                    )
                    def _():
                        wait_out(p)
                else:
                    wait_out(p)

                def b16_body(j, c2):
                    idxv = irs[ts][pl.ds(bblk * BBLK + j * L, L)]
                    for c in range(CS):
                        cv = jnp.full((L,), c, jnp.int32)
                        val = plsc.load_gather(slab_v, [idxv, cv])
                        ovs[p][c, pl.ds(j * L, L)] = val
                    return c2

                lax.fori_loop(0, NB16, b16_body, 0)
                fire_out(p, r0, bblk * BBLK)
            return carry

        lax.fori_loop(0, NBBLK // 2, bpair, 0)

    def sub_body(sub, carry):
        c0 = pl.multiple_of(
            jnp.minimum(c0b + sub * CS, VOCAB - CS), 8
        )
        fire_idx(0, t0)
        pltpu.sync_copy(table_hbm.at[:, pl.ds(c0, CS)], slab_v)

        def tpair(i, carry2):
            tA = t0 + 2 * i
            wait_idx(0)
            fire_idx(1, tA + 1)
            process_t(0, tA, c0, jnp.logical_and(i == 0, sub == 0))
            wait_idx(1)

            @pl.when(i < TPG // 2 - 1)
            def _():
                fire_idx(0, tA + 2)

            process_t(1, tA + 1, c0, jnp.bool_(False))
            return carry2

        lax.fori_loop(0, TPG // 2, tpair, 0)
        return carry

    lax.fori_loop(0, NSUB, sub_body, 0)
    wait_out(0)
    wait_out(1)

    b0 = wid * LB
    pltpu.sync_copy(idx_hbm.at[:, pl.ds(b0, LB)], idxs_v)

    def fire_tgt(slot, bb):
        pltpu.async_copy(
            tgtf_hbm.at[pl.ds((b0 + bb) * T_DIM, T_DIM)],
            tgs[slot].at[pl.ds(0, T_DIM)], tsems[slot],
        )

    def wait_tgt(slot):
        pltpu.make_async_copy(
            tgtf_hbm.at[pl.ds(0, T_DIM)],
            tgs[slot].at[pl.ds(0, T_DIM)], tsems[slot],
        ).wait()

    def fire_pairs(slot):
        pltpu.async_copy(tablef_hbm.at[pas[slot]], vas[slot], psems[slot])
        pltpu.async_copy(tablef_hbm.at[pbs[slot]], vbs[slot], psems[slot])

    def wait_pairs(slot):
        pltpu.make_async_copy(tablef_hbm.at[pas[slot]], vas[slot],
                              psems[slot]).wait()
        pltpu.make_async_copy(tablef_hbm.at[pbs[slot]], vbs[slot],
                              psems[slot]).wait()

    def group_idx(bb, j):
        tv = jnp.minimum(iota16 + j * L, T_DIM - 1)
        bbv = jnp.zeros((L,), jnp.int32) + bb
        return plsc.load_gather(idxs_v, [tv, bbv])

    def make_pairs(s, bb):
        for j in range(NG):
            idxv = group_idx(bb, j)
            tgtv = tgs[s][pl.ds(j * L, L)]
            pair = idxv * VOCAB + tgtv
            if j == NG - 1:
                pair = jnp.where(iota16 < T_DIM - (NG - 1) * L, pair, 0)
            if j < NGA:
                pas[s][pl.ds(j * L, L)] = pair
            else:
                pbs[s][pl.ds((j - NGA) * L, L)] = pair

    def acc_b(s, bb, acc):
        for j in range(NG):
            idxv = group_idx(bb, j)
            lsev = plsc.load_gather(lse_v, [idxv])
            if j < NGA:
                valv = vas[s][pl.ds(j * L, L)]
            else:
                valv = vbs[s][pl.ds((j - NGA) * L, L)]
            contrib = lsev - valv
            if j == NG - 1:
                contrib = jnp.where(iota16 < T_DIM - (NG - 1) * L,
                                    contrib, 0.0)
            acc = acc + contrib
        return acc

    fire_tgt(0, 0)
    fire_tgt(1, 1)

    def loss_body(i, acc):
        for s in (0, 1):
            bb = 2 * i + s
            wait_tgt(s)
            make_pairs(s, bb)

            @pl.when(bb < LB - 2)
            def _():
                fire_tgt(s, bb + 2)

            if s == 1:
                wait_pairs(0)
                acc = acc_b(0, 2 * i, acc)
            else:
                @pl.when(i > 0)
                def _():
                    wait_pairs(1)
                acc2 = acc_b(1, jnp.maximum(2 * i - 1, 0), acc)
                acc = jnp.where(i > 0, acc2, acc)
            fire_pairs(s)
        return acc

    acc = lax.fori_loop(0, LB // 2, loss_body, jnp.zeros((L,), jnp.float32))
    wait_pairs(1)
    acc = acc_b(1, LB - 1, acc)

    acc_v[...] = acc
    pltpu.sync_copy(acc_v, part_hbm.at[wid])


def kernel(index, targets, token_embedding_table):
    idx = index.astype(jnp.int32)
    tgt_flat = targets.reshape(-1).astype(jnp.int32)
    table = token_embedding_table

    lse = pl.pallas_call(
        _lse_body,
        out_shape=jax.ShapeDtypeStruct((VOCAB, 1), jnp.float32),
    )(table).reshape(VOCAB)

    tablef = jnp.concatenate([table.reshape(-1), jnp.zeros((8,), jnp.float32)])
    out2d, parts = _sc_gather_loss(idx, tgt_flat, table, tablef, lse)
    logits = out2d.reshape(T_DIM, VOCAB, B_DIM).transpose(2, 0, 1)
    loss = jnp.sum(parts) / jnp.float32(NTOK * T_DIM)
    return logits, loss


# speedup vs baseline: 2.7713x; 2.7713x over previous
import functools

import jax
import jax.numpy as jnp
from jax import lax
from jax.experimental import pallas as pl
from jax.experimental.pallas import tpu as pltpu
from jax.experimental.pallas import tpu_sc as plsc

VOCAB = 1000
T_DIM = 200
B_DIM = 4096
NTOK = T_DIM * B_DIM

NC, NS, L = 2, 16, 16
NW = NC * NS
TPW = NTOK // NW

KROW = 16
NBUF = 4
KAHEAD = 2
NCHUNK = TPW // KROW
NROUND = NCHUNK // NBUF

LB = B_DIM // NW
NG = 13
NGA = 8


def _lse_body(t_ref, o_ref):
    x = t_ref[...]
    m = jnp.max(x, axis=1, keepdims=True)
    o_ref[...] = m + jnp.log(jnp.sum(jnp.exp(x - m), axis=1, keepdims=True))


_MESH = plsc.VectorSubcoreMesh(
    core_axis_name="c", subcore_axis_name="s", num_cores=NC, num_subcores=NS
)


@functools.partial(
    pl.kernel,
    out_type=[
        jax.ShapeDtypeStruct((NTOK, VOCAB), jnp.float32),
        jax.ShapeDtypeStruct((NW, L), jnp.float32),
    ],
    mesh=_MESH,
    compiler_params=pltpu.CompilerParams(
        needs_layout_passes=False, use_tc_tiling_on_sc=False
    ),
    scratch_types=[
        pltpu.VMEM((TPW,), jnp.int32),
        pltpu.VMEM((VOCAB,), jnp.float32),
        pltpu.VMEM((NG * L,), jnp.int32),
        pltpu.VMEM((NG * L,), jnp.int32),
        pltpu.VMEM((NGA * L,), jnp.int32),
        pltpu.VMEM((NGA * L,), jnp.int32),
        pltpu.VMEM(((NG - NGA) * L,), jnp.int32),
        pltpu.VMEM(((NG - NGA) * L,), jnp.int32),
        pltpu.VMEM((NGA * L,), jnp.float32),
        pltpu.VMEM((NGA * L,), jnp.float32),
        pltpu.VMEM(((NG - NGA) * L,), jnp.float32),
        pltpu.VMEM(((NG - NGA) * L,), jnp.float32),
        pltpu.VMEM((L,), jnp.float32),
        pltpu.VMEM((KROW, VOCAB), jnp.float32),
        pltpu.VMEM((KROW, VOCAB), jnp.float32),
        pltpu.VMEM((KROW, VOCAB), jnp.float32),
        pltpu.VMEM((KROW, VOCAB), jnp.float32),
        pltpu.SemaphoreType.DMA,
        pltpu.SemaphoreType.DMA,
        pltpu.SemaphoreType.DMA,
        pltpu.SemaphoreType.DMA,
        pltpu.SemaphoreType.DMA,
        pltpu.SemaphoreType.DMA,
        pltpu.SemaphoreType.DMA,
        pltpu.SemaphoreType.DMA,
        pltpu.SemaphoreType.DMA,
        pltpu.SemaphoreType.DMA,
        pltpu.SemaphoreType.DMA,
        pltpu.SemaphoreType.DMA,
    ],
)
def _sc_gather_loss(idx_hbm, tgtf_hbm, table_hbm, tablef_hbm, lse_hbm,
                    out_hbm, part_hbm,
                    idxs_v, lse_v, tg0, tg1, pa0, pa1, pb0, pb1,
                    va0, va1, vb0, vb1, acc_v,
                    rows0, rows1, rows2, rows3,
                    gsem0, gsem1, gsem2, gsem3, osem0, osem1, osem2, osem3,
                    tsem0, tsem1, psem0, psem1):
    wid = lax.axis_index("s") * NC + lax.axis_index("c")
    tok0 = wid * TPW

    rows = (rows0, rows1, rows2, rows3)
    gsems = (gsem0, gsem1, gsem2, gsem3)
    osems = (osem0, osem1, osem2, osem3)
    tgs, tsems = (tg0, tg1), (tsem0, tsem1)
    pas, pbs = (pa0, pa1), (pb0, pb1)
    vas, vbs = (va0, va1), (vb0, vb1)
    psems = (psem0, psem1)
    iota16 = lax.iota(jnp.int32, L)

    pltpu.sync_copy(lse_hbm, lse_v)
    pltpu.sync_copy(idx_hbm.at[pl.ds(tok0, TPW)], idxs_v)

    def fire_g(s, c):
        off = pl.multiple_of(c * KROW, 8)
        pltpu.async_copy(
            table_hbm.at[idxs_v.at[pl.ds(off, KROW)]], rows[s], gsems[s]
        )

    def wait_g(s):
        pltpu.make_async_copy(
            table_hbm.at[pl.ds(0, KROW)], rows[s], gsems[s]
        ).wait()

    def fire_o(s, c):
        off = pl.multiple_of(c * KROW, 8)
        pltpu.async_copy(
            rows[s], out_hbm.at[pl.ds(tok0 + off, KROW)], osems[s]
        )

    def wait_o(s):
        pltpu.make_async_copy(
            rows[s], out_hbm.at[pl.ds(0, KROW)], osems[s]
        ).wait()

    for s in range(KAHEAD):
        fire_g(s, s)

    def round_body(r, carry):
        for s in range(NBUF):
            c = r * NBUF + s
            wait_g(s)
            fire_o(s, c)
            cf = c + KAHEAD
            sf = (s + KAHEAD) % NBUF

            @pl.when(cf >= NBUF)
            def _():
                wait_o(sf)

            @pl.when(cf < NCHUNK)
            def _():
                fire_g(sf, cf)
        return carry

    lax.fori_loop(0, NROUND, round_body, 0)
    for s in range(KAHEAD):
        wait_o((NCHUNK - KAHEAD + s) % NBUF)

    b0 = wid * LB

    def fire_tgt(slot, bb):
        pltpu.async_copy(
            tgtf_hbm.at[pl.ds((b0 + bb) * T_DIM, T_DIM)],
            tgs[slot].at[pl.ds(0, T_DIM)], tsems[slot],
        )

    def wait_tgt(slot):
        pltpu.make_async_copy(
            tgtf_hbm.at[pl.ds(0, T_DIM)],
            tgs[slot].at[pl.ds(0, T_DIM)], tsems[slot],
        ).wait()

    def fire_pairs(slot):
        pltpu.async_copy(tablef_hbm.at[pas[slot]], vas[slot], psems[slot])
        pltpu.async_copy(tablef_hbm.at[pbs[slot]], vbs[slot], psems[slot])

    def wait_pairs(slot):
        pltpu.make_async_copy(tablef_hbm.at[pas[slot]], vas[slot],
                              psems[slot]).wait()
        pltpu.make_async_copy(tablef_hbm.at[pbs[slot]], vbs[slot],
                              psems[slot]).wait()

    def group_idx(bb, j):
        tv = jnp.minimum(iota16 + j * L, T_DIM - 1)
        return plsc.load_gather(idxs_v, [bb * T_DIM + tv])

    def make_pairs(s, bb):
        for j in range(NG):
            idxv = group_idx(bb, j)
            tgtv = tgs[s][pl.ds(j * L, L)]
            pair = idxv * VOCAB + tgtv
            if j == NG - 1:
                pair = jnp.where(iota16 < T_DIM - (NG - 1) * L, pair, 0)
            if j < NGA:
                pas[s][pl.ds(j * L, L)] = pair
            else:
                pbs[s][pl.ds((j - NGA) * L, L)] = pair

    def acc_b(s, bb, acc):
        for j in range(NG):
            idxv = group_idx(bb, j)
            lsev = plsc.load_gather(lse_v, [idxv])
            if j < NGA:
                valv = vas[s][pl.ds(j * L, L)]
            else:
                valv = vbs[s][pl.ds((j - NGA) * L, L)]
            contrib = lsev - valv
            if j == NG - 1:
                contrib = jnp.where(iota16 < T_DIM - (NG - 1) * L,
                                    contrib, 0.0)
            acc = acc + contrib
        return acc

    fire_tgt(0, 0)
    fire_tgt(1, 1)

    def loss_body(i, acc):
        for s in (0, 1):
            bb = 2 * i + s
            wait_tgt(s)
            make_pairs(s, bb)

            @pl.when(bb < LB - 2)
            def _():
                fire_tgt(s, bb + 2)

            if s == 1:
                wait_pairs(0)
                acc = acc_b(0, 2 * i, acc)
            else:
                @pl.when(i > 0)
                def _():
                    wait_pairs(1)
                acc2 = acc_b(1, jnp.maximum(2 * i - 1, 0), acc)
                acc = jnp.where(i > 0, acc2, acc)
            fire_pairs(s)
        return acc

    acc = lax.fori_loop(0, LB // 2, loss_body, jnp.zeros((L,), jnp.float32))
    wait_pairs(1)
    acc = acc_b(1, LB - 1, acc)

    acc_v[...] = acc
    pltpu.sync_copy(acc_v, part_hbm.at[wid])


def kernel(index, targets, token_embedding_table):
    idxbt = jnp.transpose(index, (1, 0)).reshape(-1).astype(jnp.int32)
    tgt_flat = targets.reshape(-1).astype(jnp.int32)
    table = token_embedding_table

    lse = pl.pallas_call(
        _lse_body,
        out_shape=jax.ShapeDtypeStruct((VOCAB, 1), jnp.float32),
    )(table).reshape(VOCAB)

    tablef = jnp.concatenate([table.reshape(-1), jnp.zeros((8,), jnp.float32)])
    out2d, parts = _sc_gather_loss(idxbt, tgt_flat, table, tablef, lse)
    logits = out2d.reshape(B_DIM, T_DIM, VOCAB)
    loss = jnp.sum(parts) / jnp.float32(NTOK * T_DIM)
    return logits, loss
